# baseline (speedup 1.0000x reference)
import functools

import jax
import jax.numpy as jnp
from jax import lax
from jax.experimental import pallas as pl
from jax.experimental.pallas import tpu as pltpu
from jax.experimental.pallas import tpu_sc as plsc

_NC = 2
_NS = 16
_L = 16
_NW = _NC * _NS

_SC_PARAMS = pltpu.CompilerParams(
    needs_layout_passes=False, use_tc_tiling_on_sc=False
)


@functools.lru_cache(maxsize=None)
def _build_transpose(V, D, C):
    assert D == 2 * _L
    n_blocks = -(-V // C)
    assert n_blocks * C == V
    per_w = -(-n_blocks // _NW)
    scale = jnp.float32(float(D) ** 0.5)

    mesh = plsc.VectorSubcoreMesh(core_axis_name="c", subcore_axis_name="s")

    @functools.partial(
        pl.kernel,
        mesh=mesh,
        out_type=jax.ShapeDtypeStruct((V, D), jnp.float32),
        compiler_params=_SC_PARAMS,
        scratch_types=[
            pltpu.VMEM((D, C + 1), jnp.float32),
            pltpu.VMEM((D, C + 1), jnp.float32),
            pltpu.VMEM((C, D), jnp.float32),
            pltpu.VMEM((C, D), jnp.float32),
            pltpu.SemaphoreType.DMA,
            pltpu.SemaphoreType.DMA,
            pltpu.SemaphoreType.DMA,
            pltpu.SemaphoreType.DMA,
        ],
    )
    def k(src_hbm, out_hbm, in0, in1, tt0, tt1, gsem0, gsem1, wsem0, wsem1):
        wid = lax.axis_index("s") * _NC + lax.axis_index("c")
        ins = (in0, in1)
        tts = (tt0, tt1)
        gsems = (gsem0, gsem1)
        wsems = (wsem0, wsem1)
        lane = lax.iota(jnp.int32, _L)

        def blk(t):
            b = wid * per_w + t
            return jnp.where(b < n_blocks, b, b - n_blocks)

        def start_read(t, s):
            c0 = blk(t) * C
            pltpu.async_copy(
                src_hbm.at[:, pl.ds(c0, C)], ins[s].at[:, pl.ds(0, C)], gsems[s]
            )

        def wait_read(s):
            pltpu.make_async_copy(
                src_hbm.at[:, pl.ds(0, C)], ins[s].at[:, pl.ds(0, C)], gsems[s]
            ).wait()

        def start_write(t, s):
            c0 = blk(t) * C
            pltpu.async_copy(tts[s], out_hbm.at[pl.ds(c0, C)], wsems[s])

        def wait_write(s):
            pltpu.make_async_copy(
                tts[s], out_hbm.at[pl.ds(0, C)], wsems[s]
            ).wait()

        def compute(t, s):
            inb = ins[s]
            tt = tts[s]

            def col(c, _):
                cvec = jnp.full((_L,), c, jnp.int32)
                lo = plsc.load_gather(inb, [lane, cvec])
                hi = plsc.load_gather(inb, [_L + lane, cvec])
                tt[c, pl.ds(0, _L)] = lo * scale
                tt[c, pl.ds(_L, _L)] = hi * scale
                return 0

            lax.fori_loop(0, C, col, 0, unroll=8)

            @pl.when(blk(t) == 0)
            def _():
                zeros = jnp.zeros((_L,), jnp.float32)
                tt[0, pl.ds(0, _L)] = zeros
                tt[0, pl.ds(_L, _L)] = zeros

        start_read(0, 0)
        start_read(1, 1)
        for s in (0, 1):
            wait_read(s)
            compute(s, s)
            start_read(2 + s, s)
            start_write(s, s)

        def pair(i, _):
            t0 = 2 * i
            for s in (0, 1):
                t = t0 + s
                wait_read(s)
                wait_write(s)
                compute(t, s)
                start_read(t + 2, s)
                start_write(t, s)
            return 0

        lax.fori_loop(1, per_w // 2 - 1, pair, 0)

        for s in (0, 1):
            t = per_w - 2 + s
            wait_read(s)
            wait_write(s)
            compute(t, s)
            start_write(t, s)
        for s in (0, 1):
            wait_write(s)

    return k


@functools.lru_cache(maxsize=None)
def _build_gather(N, I, V, D, K):
    assert D == 2 * _L
    B = N * I
    per_w = B // _NW
    assert per_w * _NW == B
    units_i = I // K
    assert units_i * K == I
    units = N * units_i
    per_w_units = units // _NW
    assert per_w_units * _NW == units
    assert per_w_units >= 4 and per_w_units % 2 == 0

    mesh = plsc.VectorSubcoreMesh(core_axis_name="c", subcore_axis_name="s")

    @functools.partial(
        pl.kernel,
        mesh=mesh,
        out_type=jax.ShapeDtypeStruct((N, D, I), jnp.float32),
        compiler_params=_SC_PARAMS,
        scratch_types=[
            pltpu.VMEM((per_w,), jnp.int32),
            pltpu.VMEM((K, D), jnp.float32),
            pltpu.VMEM((K, D), jnp.float32),
            pltpu.VMEM((D, K + 1), jnp.float32),
            pltpu.VMEM((D, K + 1), jnp.float32),
            pltpu.SemaphoreType.DMA,
            pltpu.SemaphoreType.DMA,
            pltpu.SemaphoreType.DMA,
            pltpu.SemaphoreType.DMA,
        ],
    )
    def k(idx_hbm, table_hbm, out_hbm, idx_v, raw0, raw1, tr0, tr1,
          gsem0, gsem1, wsem0, wsem1):
        wid = lax.axis_index("s") * _NC + lax.axis_index("c")
        base_u = wid * per_w_units
        pltpu.sync_copy(idx_hbm.at[pl.ds(wid * per_w, per_w)], idx_v)

        raws = (raw0, raw1)
        trs = (tr0, tr1)
        gsems = (gsem0, gsem1)
        wsems = (wsem0, wsem1)
        lane = lax.iota(jnp.int32, _L)
        u_lo = lane
        u_hi = _L + lane

        def start_gather(t, b):
            pltpu.async_copy(
                table_hbm.at[idx_v.at[pl.ds(t * K, K)]], raws[b], gsems[b]
            )

        def wait_gather(b):
            pltpu.make_async_copy(
                table_hbm.at[idx_v.at[pl.ds(0, K)]], raws[b], gsems[b]
            ).wait()

        def start_write(t, b):
            u = base_u + t
            j = u // units_i
            i0 = (u % units_i) * K
            pltpu.async_copy(
                trs[b].at[:, pl.ds(0, K)], out_hbm.at[j, :, pl.ds(i0, K)],
                wsems[b],
            )

        def wait_write(b):
            pltpu.make_async_copy(
                trs[b].at[:, pl.ds(0, K)], out_hbm.at[0, :, pl.ds(0, K)],
                wsems[b],
            ).wait()

        def compute(b):
            raw = raws[b]
            tr = trs[b]

            def trow(r, _):
                rcol = jnp.full((_L,), r, jnp.int32)
                plsc.store_scatter(tr, [u_lo, rcol], raw[r, pl.ds(0, _L)])
                plsc.store_scatter(tr, [u_hi, rcol], raw[r, pl.ds(_L, _L)])
                return 0

            lax.fori_loop(0, K, trow, 0, unroll=8)

        start_gather(0, 0)
        start_gather(1, 1)
        for b in (0, 1):
            wait_gather(b)
            compute(b)
            start_gather(2 + b, b)
            start_write(b, b)

        def pair(i, _):
            t0 = 2 * i
            for b in (0, 1):
                t = t0 + b
                wait_gather(b)
                wait_write(b)
                compute(b)
                start_gather(t + 2, b)
                start_write(t, b)
            return 0

        lax.fori_loop(1, per_w_units // 2 - 1, pair, 0)

        for b in (0, 1):
            t = per_w_units - 2 + b
            wait_gather(b)
            wait_write(b)
            compute(b)
            start_write(t, b)
        for b in (0, 1):
            wait_write(b)

    return k


def kernel(inputs, lookup_table):
    V, D = lookup_table.shape
    I, N = inputs.shape
    table_fm = lookup_table.T
    scaled = _build_transpose(V, D, 800)(table_fm)
    idx = inputs.T.reshape(-1).astype(jnp.int32)
    out = _build_gather(N, I, V, D, 512)(idx, scaled)
    return jnp.transpose(out, (2, 0, 1))


# speedup vs baseline: 3.6531x; 3.6531x over previous
import functools

import jax
import jax.numpy as jnp
from jax import lax
from jax.experimental import pallas as pl
from jax.experimental.pallas import tpu as pltpu
from jax.experimental.pallas import tpu_sc as plsc

_NC = 2
_NS = 16
_L = 16
_NW = _NC * _NS

_ROW_PAD = 128


@functools.lru_cache(maxsize=None)
def _build_gather(N, I, VP, D, K):
    assert D == 2 * _L
    B = N * I
    per_w = B // _NW
    assert per_w * _NW == B
    units_i = I // K
    assert units_i * K == I
    units = N * units_i
    per_w_units = units // _NW
    assert per_w_units * _NW == units
    assert per_w_units >= 4 and per_w_units % 2 == 0
    scale = jnp.float32(float(D) ** 0.5)

    mesh = plsc.VectorSubcoreMesh(core_axis_name="c", subcore_axis_name="s")

    @functools.partial(
        pl.kernel,
        mesh=mesh,
        out_type=jax.ShapeDtypeStruct((N, D, I), jnp.float32),
        compiler_params=pltpu.CompilerParams(
            needs_layout_passes=False, use_tc_tiling_on_sc=False
        ),
        scratch_types=[
            pltpu.VMEM((per_w,), jnp.int32),
            pltpu.VMEM((K, D), jnp.float32),
            pltpu.VMEM((K, D), jnp.float32),
            pltpu.VMEM((D, K + 1), jnp.float32),
            pltpu.VMEM((D, K + 1), jnp.float32),
            pltpu.SemaphoreType.DMA,
            pltpu.SemaphoreType.DMA,
            pltpu.SemaphoreType.DMA,
            pltpu.SemaphoreType.DMA,
        ],
    )
    def k(idx_hbm, table_hbm, out_hbm, idx_v, raw0, raw1, tr0, tr1,
          gsem0, gsem1, wsem0, wsem1):
        wid = lax.axis_index("s") * _NC + lax.axis_index("c")
        base_u = wid * per_w_units
        pltpu.sync_copy(idx_hbm.at[pl.ds(wid * per_w, per_w)], idx_v)

        raws = (raw0, raw1)
        trs = (tr0, tr1)
        gsems = (gsem0, gsem1)
        wsems = (wsem0, wsem1)
        lane = lax.iota(jnp.int32, _L)
        u_lo = lane
        u_hi = _L + lane

        def start_gather(t, b):
            pltpu.async_copy(
                table_hbm.at[idx_v.at[pl.ds(t * K, K)]], raws[b], gsems[b]
            )

        def wait_gather(b):
            pltpu.make_async_copy(
                table_hbm.at[idx_v.at[pl.ds(0, K)]], raws[b], gsems[b]
            ).wait()

        def start_write(t, b):
            u = base_u + t
            j = u // units_i
            i0 = (u % units_i) * K
            pltpu.async_copy(
                trs[b].at[:, pl.ds(0, K)], out_hbm.at[j, :, pl.ds(i0, K)],
                wsems[b],
            )

        def wait_write(b):
            pltpu.make_async_copy(
                trs[b].at[:, pl.ds(0, K)], out_hbm.at[0, :, pl.ds(0, K)],
                wsems[b],
            ).wait()

        def compute(b):
            raw = raws[b]
            tr = trs[b]

            def trow(r, _):
                rcol = jnp.full((_L,), r, jnp.int32)
                plsc.store_scatter(tr, [u_lo, rcol], raw[r, pl.ds(0, _L)] * scale)
                plsc.store_scatter(tr, [u_hi, rcol], raw[r, pl.ds(_L, _L)] * scale)
                return 0

            lax.fori_loop(0, K, trow, 0, unroll=8)

        start_gather(0, 0)
        start_gather(1, 1)
        for b in (0, 1):
            wait_gather(b)
            compute(b)
            start_gather(2 + b, b)
            start_write(b, b)

        def pair(i, _):
            t0 = 2 * i
            for b in (0, 1):
                t = t0 + b
                wait_gather(b)
                wait_write(b)
                compute(b)
                start_gather(t + 2, b)
                start_write(t, b)
            return 0

        lax.fori_loop(1, per_w_units // 2 - 1, pair, 0)

        for b in (0, 1):
            t = per_w_units - 2 + b
            wait_gather(b)
            wait_write(b)
            compute(b)
            start_write(t, b)
        for b in (0, 1):
            wait_write(b)

    return k


def kernel(inputs, lookup_table):
    V, D = lookup_table.shape
    I, N = inputs.shape
    rows_per = _ROW_PAD // D
    tbl = jnp.pad(lookup_table, ((0, 0), (0, _ROW_PAD - D)))
    tbl = tbl.reshape(rows_per * V, D)
    idx = inputs.T.reshape(-1).astype(jnp.int32)
    idx = idx * rows_per + (idx == 0).astype(jnp.int32)
    out = _build_gather(N, I, rows_per * V, D, 512)(idx, tbl)
    return jnp.transpose(out, (2, 0, 1))


# speedup vs baseline: 5.4390x; 1.4889x over previous
import functools

import jax
import jax.numpy as jnp
from jax import lax
from jax.experimental import pallas as pl
from jax.experimental.pallas import tpu as pltpu
from jax.experimental.pallas import tpu_sc as plsc

_NC = 2
_NS = 16
_L = 16
_NW = _NC * _NS

_ROW_PAD = 128


@functools.lru_cache(maxsize=None)
def _build_gather(N, I, VP, D, K):
    assert D == 2 * _L
    B = N * I
    per_w = B // _NW
    assert per_w * _NW == B
    units_i = I // K
    assert units_i * K == I
    units = N * units_i
    per_w_units = units // _NW
    assert per_w_units * _NW == units
    assert per_w_units >= 4 and per_w_units % 2 == 0
    assert K % 128 == 0 and D % 8 == 0
    tiles_i = K // 128
    scale = jnp.float32(float(D) ** 0.5)

    mesh = plsc.VectorSubcoreMesh(core_axis_name="c", subcore_axis_name="s")

    @functools.partial(
        pl.kernel,
        mesh=mesh,
        out_type=jax.ShapeDtypeStruct((N, D // 8, I // 128, 8, 128), jnp.float32),
        compiler_params=pltpu.CompilerParams(
            needs_layout_passes=False, use_tc_tiling_on_sc=False
        ),
        scratch_types=[
            pltpu.VMEM((per_w,), jnp.int32),
            pltpu.VMEM((K, D), jnp.float32),
            pltpu.VMEM((K, D), jnp.float32),
            pltpu.VMEM((D, K + 1), jnp.float32),
            pltpu.VMEM((D, K + 1), jnp.float32),
            pltpu.SemaphoreType.DMA,
            pltpu.SemaphoreType.DMA,
            pltpu.SemaphoreType.DMA,
            pltpu.SemaphoreType.DMA,
        ],
    )
    def k(idx_hbm, table_hbm, out_hbm, idx_v, raw0, raw1, tr0, tr1,
          gsem0, gsem1, wsem0, wsem1):
        wid = lax.axis_index("s") * _NC + lax.axis_index("c")
        base_u = wid * per_w_units
        pltpu.sync_copy(idx_hbm.at[pl.ds(wid * per_w, per_w)], idx_v)

        raws = (raw0, raw1)
        trs = (tr0, tr1)
        gsems = (gsem0, gsem1)
        wsems = (wsem0, wsem1)
        lane = lax.iota(jnp.int32, _L)
        u_lo = lane
        u_hi = _L + lane

        def start_gather(t, b):
            pltpu.async_copy(
                table_hbm.at[idx_v.at[pl.ds(t * K, K)]], raws[b], gsems[b]
            )

        def wait_gather(b):
            pltpu.make_async_copy(
                table_hbm.at[idx_v.at[pl.ds(0, K)]], raws[b], gsems[b]
            ).wait()

        def start_write(t, b):
            u = base_u + t
            j = u // units_i
            ib0 = (u % units_i) * tiles_i
            for ub in range(D // 8):
                for ib in range(tiles_i):
                    pltpu.async_copy(
                        trs[b].at[pl.ds(ub * 8, 8), pl.ds(ib * 128, 128)],
                        out_hbm.at[j, ub, ib0 + ib],
                        wsems[b],
                    )

        def wait_write(b):
            for _ in range((D // 8) * tiles_i):
                pltpu.make_async_copy(
                    trs[b].at[pl.ds(0, 8), pl.ds(0, 128)],
                    out_hbm.at[0, 0, 0],
                    wsems[b],
                ).wait()

        def compute(b):
            raw = raws[b]
            tr = trs[b]

            @plsc.parallel_loop(0, K, unroll=8)
            def trow(r):
                rcol = jnp.full((_L,), r, jnp.int32)
                plsc.store_scatter(tr, [u_lo, rcol], raw[r, pl.ds(0, _L)] * scale)
                plsc.store_scatter(tr, [u_hi, rcol], raw[r, pl.ds(_L, _L)] * scale)

        start_gather(0, 0)
        start_gather(1, 1)
        for b in (0, 1):
            wait_gather(b)
            compute(b)
            start_gather(2 + b, b)
            start_write(b, b)

        def pair(i, _):
            t0 = 2 * i
            for b in (0, 1):
                t = t0 + b
                wait_gather(b)
                wait_write(b)
                compute(b)
                start_gather(t + 2, b)
                start_write(t, b)
            return 0

        lax.fori_loop(1, per_w_units // 2 - 1, pair, 0)

        for b in (0, 1):
            t = per_w_units - 2 + b
            wait_gather(b)
            wait_write(b)
            compute(b)
            start_write(t, b)
        for b in (0, 1):
            wait_write(b)

    return k


def kernel(inputs, lookup_table):
    V, D = lookup_table.shape
    I, N = inputs.shape
    rows_per = _ROW_PAD // D
    tbl = jnp.pad(lookup_table, ((0, 0), (0, _ROW_PAD - D)))
    tbl = tbl.reshape(rows_per * V, D)
    idx = inputs.T.reshape(-1).astype(jnp.int32)
    idx = idx * rows_per + (idx == 0).astype(jnp.int32)
    out5 = _build_gather(N, I, rows_per * V, D, 512)(idx, tbl)
    out = jnp.transpose(out5, (2, 4, 0, 1, 3)).reshape(I, N, D)
    return out


# speedup vs baseline: 6.5543x; 1.2050x over previous
import functools

import jax
import jax.numpy as jnp
from jax import lax
from jax.experimental import pallas as pl
from jax.experimental.pallas import tpu as pltpu
from jax.experimental.pallas import tpu_sc as plsc

_NC = 2
_NS = 16
_L = 16
_NW = _NC * _NS

_ROW_PAD = 128


@functools.lru_cache(maxsize=None)
def _build_tile_transpose(VP, D):
    assert D == 2 * _L and VP % 128 == 0
    n_vb = VP // 128
    per_w = -(-n_vb // _NW)
    per_w += per_w % 2
    scale = jnp.float32(float(D) ** 0.5)

    mesh = plsc.VectorSubcoreMesh(core_axis_name="c", subcore_axis_name="s")

    @functools.partial(
        pl.kernel,
        mesh=mesh,
        out_type=jax.ShapeDtypeStruct((VP, D), jnp.float32),
        compiler_params=pltpu.CompilerParams(
            needs_layout_passes=False, use_tc_tiling_on_sc=False
        ),
        scratch_types=[
            pltpu.VMEM((D // 8, 8, 128), jnp.float32),
            pltpu.VMEM((D // 8, 8, 128), jnp.float32),
            pltpu.VMEM((128, D + 1), jnp.float32),
            pltpu.VMEM((128, D + 1), jnp.float32),
            pltpu.SemaphoreType.DMA,
            pltpu.SemaphoreType.DMA,
            pltpu.SemaphoreType.DMA,
            pltpu.SemaphoreType.DMA,
        ],
    )
    def k(tiles_hbm, out_hbm, in0, in1, tt0, tt1, gsem0, gsem1, wsem0, wsem1):
        wid = lax.axis_index("s") * _NC + lax.axis_index("c")
        ins = (in0, in1)
        tts = (tt0, tt1)
        gsems = (gsem0, gsem1)
        wsems = (wsem0, wsem1)
        lane = lax.iota(jnp.int32, _L)

        def blk(t):
            b = wid * per_w + t
            return jnp.where(b < n_vb, b, b - n_vb)

        def start_read(t, s):
            pltpu.async_copy(tiles_hbm.at[:, blk(t)], ins[s], gsems[s])

        def wait_read(s):
            pltpu.make_async_copy(
                tiles_hbm.at[:, 0], ins[s], gsems[s]
            ).wait()

        def start_write(t, s):
            pltpu.async_copy(
                tts[s].at[:, pl.ds(0, D)],
                out_hbm.at[pl.ds(blk(t) * 128, 128)],
                wsems[s],
            )

        def wait_write(s):
            pltpu.make_async_copy(
                tts[s].at[:, pl.ds(0, D)],
                out_hbm.at[pl.ds(0, 128)],
                wsems[s],
            ).wait()

        def compute(t, s):
            inb = ins[s]
            tt = tts[s]

            @plsc.parallel_loop(0, 128 // _L, unroll=2)
            def col(c):
                rows = c * _L + lane
                for ub in range(D // 8):
                    for us in range(8):
                        u = ub * 8 + us
                        vec = inb[ub, us, pl.ds(c * _L, _L)] * scale
                        plsc.store_scatter(
                            tt, [rows, jnp.full((_L,), u, jnp.int32)], vec
                        )

            @pl.when(blk(t) == 0)
            def _():
                zeros = jnp.zeros((_L,), jnp.float32)
                tt[0, pl.ds(0, _L)] = zeros
                tt[0, pl.ds(_L, _L)] = zeros

        start_read(0, 0)
        start_read(1, 1)
        for s in (0, 1):
            wait_read(s)
            compute(s, s)
            start_read(2 + s, s)
            start_write(s, s)

        def pair(i, _):
            t0 = 2 * i
            for s in (0, 1):
                t = t0 + s
                wait_read(s)
                wait_write(s)
                compute(t, s)
                start_read(t + 2, s)
                start_write(t, s)
            return 0

        lax.fori_loop(1, per_w // 2 - 1, pair, 0)

        for s in (0, 1):
            t = per_w - 2 + s
            wait_read(s)
            wait_write(s)
            compute(t, s)
            start_write(t, s)
        for s in (0, 1):
            wait_write(s)

    return k


@functools.lru_cache(maxsize=None)
def _build_gather(N, I, VP, D, K):
    assert D == 2 * _L
    B = N * I
    per_w = B // _NW
    assert per_w * _NW == B
    units_i = I // K
    assert units_i * K == I
    units = N * units_i
    per_w_units = units // _NW
    assert per_w_units * _NW == units
    assert per_w_units >= 4 and per_w_units % 2 == 0
    assert K % 128 == 0 and D % 8 == 0
    tiles_i = K // 128

    mesh = plsc.VectorSubcoreMesh(core_axis_name="c", subcore_axis_name="s")

    @functools.partial(
        pl.kernel,
        mesh=mesh,
        out_type=jax.ShapeDtypeStruct((N, D // 8, I // 128, 8, 128), jnp.float32),
        compiler_params=pltpu.CompilerParams(
            needs_layout_passes=False, use_tc_tiling_on_sc=False
        ),
        scratch_types=[
            pltpu.VMEM((per_w,), jnp.int32),
            pltpu.VMEM((K, D), jnp.float32),
            pltpu.VMEM((K, D), jnp.float32),
            pltpu.VMEM((D, K + 1), jnp.float32),
            pltpu.VMEM((D, K + 1), jnp.float32),
            pltpu.SemaphoreType.DMA,
            pltpu.SemaphoreType.DMA,
            pltpu.SemaphoreType.DMA,
            pltpu.SemaphoreType.DMA,
        ],
    )
    def k(idx_hbm, table_hbm, out_hbm, idx_v, raw0, raw1, tr0, tr1,
          gsem0, gsem1, wsem0, wsem1):
        wid = lax.axis_index("s") * _NC + lax.axis_index("c")
        base_u = wid * per_w_units
        pltpu.sync_copy(idx_hbm.at[pl.ds(wid * per_w, per_w)], idx_v)

        raws = (raw0, raw1)
        trs = (tr0, tr1)
        gsems = (gsem0, gsem1)
        wsems = (wsem0, wsem1)
        lane = lax.iota(jnp.int32, _L)
        u_lo = lane
        u_hi = _L + lane

        def start_gather(t, b):
            pltpu.async_copy(
                table_hbm.at[idx_v.at[pl.ds(t * K, K)]], raws[b], gsems[b]
            )

        def wait_gather(b):
            pltpu.make_async_copy(
                table_hbm.at[idx_v.at[pl.ds(0, K)]], raws[b], gsems[b]
            ).wait()

        def start_write(t, b):
            u = base_u + t
            j = u // units_i
            ib0 = (u % units_i) * tiles_i
            for ub in range(D // 8):
                for ib in range(tiles_i):
                    pltpu.async_copy(
                        trs[b].at[pl.ds(ub * 8, 8), pl.ds(ib * 128, 128)],
                        out_hbm.at[j, ub, ib0 + ib],
                        wsems[b],
                    )

        def wait_write(b):
            for _ in range((D // 8) * tiles_i):
                pltpu.make_async_copy(
                    trs[b].at[pl.ds(0, 8), pl.ds(0, 128)],
                    out_hbm.at[0, 0, 0],
                    wsems[b],
                ).wait()

        def compute(b):
            raw = raws[b]
            tr = trs[b]

            @plsc.parallel_loop(0, K, unroll=8)
            def trow(r):
                rcol = jnp.full((_L,), r, jnp.int32)
                plsc.store_scatter(tr, [u_lo, rcol], raw[r, pl.ds(0, _L)])
                plsc.store_scatter(tr, [u_hi, rcol], raw[r, pl.ds(_L, _L)])

        start_gather(0, 0)
        start_gather(1, 1)
        for b in (0, 1):
            wait_gather(b)
            compute(b)
            start_gather(2 + b, b)
            start_write(b, b)

        def pair(i, _):
            t0 = 2 * i
            for b in (0, 1):
                t = t0 + b
                wait_gather(b)
                wait_write(b)
                compute(b)
                start_gather(t + 2, b)
                start_write(t, b)
            return 0

        lax.fori_loop(1, per_w_units // 2 - 1, pair, 0)

        for b in (0, 1):
            t = per_w_units - 2 + b
            wait_gather(b)
            wait_write(b)
            compute(b)
            start_write(t, b)
        for b in (0, 1):
            wait_write(b)

    return k


def kernel(inputs, lookup_table):
    V, D = lookup_table.shape
    I, N = inputs.shape
    VP = -(-V // 128) * 128
    tbl = jnp.pad(lookup_table, ((0, VP - V), (0, 0)))
    tiles = tbl.T.reshape(D // 8, 8, VP // 128, 128).transpose(0, 2, 1, 3)
    scaled = _build_tile_transpose(VP, D)(tiles)
    idx = inputs.T.reshape(-1).astype(jnp.int32)
    out5 = _build_gather(N, I, VP, D, 512)(idx, scaled)
    out = jnp.transpose(out5, (2, 4, 0, 1, 3)).reshape(I, N, D)
    return out
